# baseline (speedup 1.0000x reference)
import functools

import jax
import jax.numpy as jnp
from jax import lax
from jax.experimental import pallas as pl
from jax.experimental.pallas import tpu as pltpu
from jax.experimental.pallas import tpu_sc as plsc

_NUM_Q = 6
_K = 1024
_D = 256
_BLK = 1024

_NC = 2
_NS = 16
_NW = _NC * _NS


def _stage_body(r_ref, cb_ref, a_ref, c_ref, idx_ref, xd_ref, counts_ref,
                loss_ref, perp_ref, *, n_blocks, n_tokens):
    i = pl.program_id(0)
    r = r_ref[...]
    cb = cb_ref[...]
    a = a_ref[...]
    c = c_ref[...]

    m = jax.lax.dot_general(r, cb, dimension_numbers=(((1,), (1,)), ((), ())),
                            preferred_element_type=jnp.float32)
    dist = a - 2.0 * m + c
    neg = -dist
    mx = jnp.max(neg, axis=-1, keepdims=True)
    iota = jax.lax.broadcasted_iota(jnp.int32, neg.shape, 1)
    idx = jnp.min(jnp.where(neg == mx, iota, _K), axis=-1)
    idx_ref[...] = idx

    onehot = (iota == idx[:, None]).astype(jnp.float32)
    xd_ref[...] = jax.lax.dot_general(
        onehot, cb, dimension_numbers=(((1,), (0,)), ((), ())),
        preferred_element_type=jnp.float32,
        precision=jax.lax.Precision.HIGHEST)
    blk_counts = jnp.sum(onehot, axis=0, keepdims=True)
    blk_loss = jnp.sum(-mx).reshape(1, 1)

    @pl.when(i == 0)
    def _init():
        counts_ref[...] = blk_counts
        loss_ref[...] = blk_loss

    @pl.when(i > 0)
    def _acc():
        counts_ref[...] += blk_counts
        loss_ref[...] += blk_loss

    @pl.when(i == n_blocks - 1)
    def _finalize():
        counts = counts_ref[...]
        prob = counts / jnp.float32(n_tokens)
        perp_ref[...] = jnp.exp(-jnp.sum(prob * jnp.log(prob + 1e-7))).reshape(1, 1)
        loss_ref[...] = loss_ref[...] / jnp.float32(n_tokens * _D)


def _stage_call(r, cb, a, c):
    b = r.shape[0]
    n_blocks = b // _BLK
    body = functools.partial(_stage_body, n_blocks=n_blocks, n_tokens=b)
    return pl.pallas_call(
        body,
        grid=(n_blocks,),
        in_specs=[
            pl.BlockSpec((_BLK, _D), lambda i: (i, 0)),
            pl.BlockSpec((_K, _D), lambda i: (0, 0)),
            pl.BlockSpec((_BLK, 1), lambda i: (i, 0)),
            pl.BlockSpec((1, _K), lambda i: (0, 0)),
        ],
        out_specs=[
            pl.BlockSpec((_BLK,), lambda i: (i,)),
            pl.BlockSpec((_BLK, _D), lambda i: (i, 0)),
            pl.BlockSpec((1, _K), lambda i: (0, 0)),
            pl.BlockSpec((1, 1), lambda i: (0, 0)),
            pl.BlockSpec((1, 1), lambda i: (0, 0)),
        ],
        out_shape=[
            jax.ShapeDtypeStruct((b,), jnp.int32),
            jax.ShapeDtypeStruct((b, _D), jnp.float32),
            jax.ShapeDtypeStruct((1, _K), jnp.float32),
            jax.ShapeDtypeStruct((1, 1), jnp.float32),
            jax.ShapeDtypeStruct((1, 1), jnp.float32),
        ],
        compiler_params=pltpu.CompilerParams(
            dimension_semantics=("arbitrary",),
        ),
    )(r, cb, a, c)


def _make_sc_gather(b):
    b_per_w = b // _NW
    mesh = plsc.VectorSubcoreMesh(core_axis_name="c", subcore_axis_name="s")

    @functools.partial(
        pl.kernel, mesh=mesh,
        out_type=jax.ShapeDtypeStruct((b, _D), jnp.float32),
        scratch_types=[
            pltpu.VMEM((b_per_w,), jnp.int32),
            pltpu.VMEM((b_per_w, _D), jnp.float32),
            pltpu.SemaphoreType.DMA,
        ],
    )
    def gather_rows(table_hbm, idx_hbm, out_hbm, idx_v, rows_v, sem):
        wid = lax.axis_index("s") * _NC + lax.axis_index("c")
        base = wid * b_per_w
        pltpu.sync_copy(idx_hbm.at[pl.ds(base, b_per_w)], idx_v)
        pltpu.async_copy(table_hbm.at[idx_v], rows_v, sem).wait()
        pltpu.sync_copy(rows_v, out_hbm.at[pl.ds(base, b_per_w)])

    return gather_rows


_sc_gather = _make_sc_gather(4096)


def kernel(x, codebooks):
    n, cdim, t = x.shape
    q_num, k_num, d = codebooks.shape
    r0 = jnp.transpose(x, (0, 2, 1)).reshape(n * t, cdim)
    c_all = jnp.sum(codebooks ** 2, axis=-1)
    r = r0
    idxs, losses, perps = [], [], []
    for q in range(q_num):
        a = jnp.sum(r ** 2, axis=-1, keepdims=True)
        idx, xd, _counts, loss, perp = _stage_call(
            r, codebooks[q], a, c_all[q][None, :])
        x_st = r + (xd - r)
        r = r - x_st
        idxs.append(idx.reshape(n, t))
        losses.append(loss[0, 0])
        perps.append(perp[0, 0])
    out = r0 - r
    quantized_out = jnp.transpose(out.reshape(n, t, cdim), (0, 2, 1))
    return (quantized_out, jnp.stack(idxs, axis=-1), jnp.stack(losses),
            jnp.stack(perps))


# speedup vs baseline: 1.2970x; 1.2970x over previous
import functools

import jax
import jax.numpy as jnp
from jax import lax
from jax.experimental import pallas as pl
from jax.experimental.pallas import tpu as pltpu
from jax.experimental.pallas import tpu_sc as plsc

_NUM_Q = 6
_K = 1024
_D = 256
_BLK = 1024

_NC = 2
_NS = 16
_NW = _NC * _NS


def _stage_body(r_ref, cb_ref, a_ref, c_ref, idx_ref, counts_ref,
                loss_ref, perp_ref, *, n_blocks, n_tokens):
    i = pl.program_id(0)
    r = r_ref[...]
    cb = cb_ref[...]
    a = a_ref[...]
    c = c_ref[...]

    m = jax.lax.dot_general(r, cb, dimension_numbers=(((1,), (1,)), ((), ())),
                            preferred_element_type=jnp.float32)
    a_col = jnp.transpose(a, (1, 0))
    dist = a_col - 2.0 * m + c
    mn = jnp.min(dist, axis=-1, keepdims=True)
    iota = jax.lax.broadcasted_iota(jnp.int32, dist.shape, 1)
    idx = jnp.min(jnp.where(dist == mn, iota, _K), axis=-1)
    idx_ref[...] = idx

    onehot = (iota == idx[:, None]).astype(jnp.float32)
    blk_counts = jnp.sum(onehot, axis=0, keepdims=True)
    blk_loss = jnp.sum(mn).reshape(1, 1)

    @pl.when(i == 0)
    def _init():
        counts_ref[...] = blk_counts
        loss_ref[...] = blk_loss

    @pl.when(i > 0)
    def _acc():
        counts_ref[...] += blk_counts
        loss_ref[...] += blk_loss

    @pl.when(i == n_blocks - 1)
    def _finalize():
        counts = counts_ref[...]
        prob = counts / jnp.float32(n_tokens)
        perp_ref[...] = jnp.exp(-jnp.sum(prob * jnp.log(prob + 1e-7))).reshape(1, 1)
        loss_ref[...] = loss_ref[...] / jnp.float32(n_tokens * _D)


def _stage_call(r, cb, a, c):
    b = r.shape[0]
    n_blocks = b // _BLK
    body = functools.partial(_stage_body, n_blocks=n_blocks, n_tokens=b)
    return pl.pallas_call(
        body,
        grid=(n_blocks,),
        in_specs=[
            pl.BlockSpec((_BLK, _D), lambda i: (i, 0)),
            pl.BlockSpec((_K, _D), lambda i: (0, 0)),
            pl.BlockSpec((1, _BLK), lambda i: (0, i)),
            pl.BlockSpec((1, _K), lambda i: (0, 0)),
        ],
        out_specs=[
            pl.BlockSpec((_BLK,), lambda i: (i,)),
            pl.BlockSpec((1, _K), lambda i: (0, 0)),
            pl.BlockSpec((1, 1), lambda i: (0, 0)),
            pl.BlockSpec((1, 1), lambda i: (0, 0)),
        ],
        out_shape=[
            jax.ShapeDtypeStruct((b,), jnp.int32),
            jax.ShapeDtypeStruct((1, _K), jnp.float32),
            jax.ShapeDtypeStruct((1, 1), jnp.float32),
            jax.ShapeDtypeStruct((1, 1), jnp.float32),
        ],
        compiler_params=pltpu.CompilerParams(
            dimension_semantics=("arbitrary",),
        ),
    )(r, cb, a, c)


def _make_sc_gather(b):
    b_per_w = b // _NW
    mesh = plsc.VectorSubcoreMesh(core_axis_name="c", subcore_axis_name="s")

    @functools.partial(
        pl.kernel, mesh=mesh,
        out_type=jax.ShapeDtypeStruct((b, _D), jnp.float32),
        scratch_types=[
            pltpu.VMEM((b_per_w,), jnp.int32),
            pltpu.VMEM((b_per_w, _D), jnp.float32),
            pltpu.SemaphoreType.DMA,
        ],
    )
    def gather_rows(table_hbm, idx_hbm, out_hbm, idx_v, rows_v, sem):
        wid = lax.axis_index("s") * _NC + lax.axis_index("c")
        base = wid * b_per_w
        pltpu.sync_copy(idx_hbm.at[pl.ds(base, b_per_w)], idx_v)
        pltpu.async_copy(table_hbm.at[idx_v], rows_v, sem).wait()
        pltpu.sync_copy(rows_v, out_hbm.at[pl.ds(base, b_per_w)])

    return gather_rows


_sc_gather = _make_sc_gather(4096)


def kernel(x, codebooks):
    n, cdim, t = x.shape
    q_num, k_num, d = codebooks.shape
    r0 = jnp.transpose(x, (0, 2, 1)).reshape(n * t, cdim)
    c_all = jnp.sum(codebooks ** 2, axis=-1)
    r = r0
    idxs, losses, perps = [], [], []
    for q in range(q_num):
        a = jnp.sum(r ** 2, axis=-1, keepdims=True)
        idx, _counts, loss, perp = _stage_call(
            r, codebooks[q], jnp.transpose(a, (1, 0)), c_all[q][None, :])
        xd = _sc_gather(codebooks[q], idx)
        x_st = r + (xd - r)
        r = r - x_st
        idxs.append(idx.reshape(n, t))
        losses.append(loss[0, 0])
        perps.append(perp[0, 0])
    out = r0 - r
    quantized_out = jnp.transpose(out.reshape(n, t, cdim), (0, 2, 1))
    return (quantized_out, jnp.stack(idxs, axis=-1), jnp.stack(losses),
            jnp.stack(perps))
